# baseline (speedup 1.0000x reference)
import jax
import jax.numpy as jnp
from jax import lax
from jax.experimental import pallas as pl
from jax.experimental.pallas import tpu as pltpu, tpu_sc as plsc

NODE_NUM = 100
HIDDEN_SIZE = 4096
TOTAL = NODE_NUM * HIDDEN_SIZE

NUM_CORES = 2
NUM_SUBCORES = 16
NUM_WORKERS = NUM_CORES * NUM_SUBCORES
CHUNK = TOTAL // NUM_WORKERS


def _copy_body(in_hbm, out_hbm):
    wid = lax.axis_index("s") * NUM_CORES + lax.axis_index("c")
    base = wid * CHUNK
    pltpu.sync_copy(in_hbm.at[pl.ds(base, CHUNK)], out_hbm.at[pl.ds(base, CHUNK)])


def kernel(node_table):
    flat = node_table.reshape(TOTAL)
    mesh = plsc.VectorSubcoreMesh(core_axis_name="c", subcore_axis_name="s")
    out = pl.kernel(
        _copy_body,
        out_type=jax.ShapeDtypeStruct((TOTAL,), jnp.float32),
        mesh=mesh,
    )(flat)
    return out.reshape(NODE_NUM, HIDDEN_SIZE)


# speedup vs baseline: 1.0136x; 1.0136x over previous
import jax
import jax.numpy as jnp
from jax import lax
from jax.experimental import pallas as pl
from jax.experimental.pallas import tpu as pltpu, tpu_sc as plsc

NODE_NUM = 100
HIDDEN_SIZE = 4096
TOTAL = NODE_NUM * HIDDEN_SIZE

NUM_CORES = 2
NUM_SUBCORES = 16
NUM_WORKERS = NUM_CORES * NUM_SUBCORES
CHUNK = TOTAL // NUM_WORKERS


HALF = TOTAL // NUM_CORES


def _copy_body(in_hbm, out_hbm):
    cid = lax.axis_index("c")
    base = cid * HALF
    pltpu.sync_copy(in_hbm.at[pl.ds(base, HALF)], out_hbm.at[pl.ds(base, HALF)])


def kernel(node_table):
    flat = node_table.reshape(TOTAL)
    mesh = plsc.ScalarSubcoreMesh(axis_name="c", num_cores=NUM_CORES)
    out = pl.kernel(
        _copy_body,
        out_type=jax.ShapeDtypeStruct((TOTAL,), jnp.float32),
        mesh=mesh,
    )(flat)
    return out.reshape(NODE_NUM, HIDDEN_SIZE)


# speedup vs baseline: 1.3804x; 1.3619x over previous
import jax
import jax.numpy as jnp
from jax.experimental import pallas as pl
from jax.experimental.pallas import tpu as pltpu

NODE_NUM = 100
HIDDEN_SIZE = 4096


def _dma_body(in_hbm, out_hbm, sem):
    copy = pltpu.make_async_copy(in_hbm, out_hbm, sem)
    copy.start()
    copy.wait()


def kernel(node_table):
    return pl.pallas_call(
        _dma_body,
        out_shape=jax.ShapeDtypeStruct((NODE_NUM, HIDDEN_SIZE), jnp.float32),
        in_specs=[pl.BlockSpec(memory_space=pl.ANY)],
        out_specs=pl.BlockSpec(memory_space=pl.ANY),
        scratch_shapes=[pltpu.SemaphoreType.DMA],
    )(node_table)


# speedup vs baseline: 12.8744x; 9.3264x over previous
import jax
import jax.numpy as jnp
from jax.experimental import pallas as pl
from jax.experimental.pallas import tpu as pltpu

NODE_NUM = 100
HIDDEN_SIZE = 4096
BLOCK_COLS = 512
GRID = HIDDEN_SIZE // BLOCK_COLS


def _copy_block(in_ref, out_ref):
    out_ref[...] = in_ref[...]


def kernel(node_table):
    return pl.pallas_call(
        _copy_block,
        out_shape=jax.ShapeDtypeStruct((NODE_NUM, HIDDEN_SIZE), jnp.float32),
        grid=(GRID,),
        in_specs=[
            pl.BlockSpec((NODE_NUM, BLOCK_COLS), lambda i: (0, i)),
        ],
        out_specs=pl.BlockSpec((NODE_NUM, BLOCK_COLS), lambda i: (0, i)),
    )(node_table)


# speedup vs baseline: 25.6011x; 1.9885x over previous
import jax
import jax.numpy as jnp
from jax.experimental import pallas as pl
from jax.experimental.pallas import tpu as pltpu

NODE_NUM = 100
HIDDEN_SIZE = 4096
BLOCK_COLS = 4096
GRID = HIDDEN_SIZE // BLOCK_COLS


def _copy_block(in_ref, out_ref):
    out_ref[...] = in_ref[...]


def kernel(node_table):
    return pl.pallas_call(
        _copy_block,
        out_shape=jax.ShapeDtypeStruct((NODE_NUM, HIDDEN_SIZE), jnp.float32),
        grid=(GRID,),
        in_specs=[
            pl.BlockSpec((NODE_NUM, BLOCK_COLS), lambda i: (0, i)),
        ],
        out_specs=pl.BlockSpec((NODE_NUM, BLOCK_COLS), lambda i: (0, i)),
    )(node_table)


# speedup vs baseline: 29.5708x; 1.1551x over previous
import jax
import jax.numpy as jnp
from jax.experimental import pallas as pl
from jax.experimental.pallas import tpu as pltpu

NODE_NUM = 100
HIDDEN_SIZE = 4096
NCHUNK = 4
CHUNK_COLS = HIDDEN_SIZE // NCHUNK


def _dma_body(in_hbm, out_hbm, buf, insem, outsem):
    for c in range(NCHUNK):
        pltpu.make_async_copy(
            in_hbm.at[:, pl.ds(c * CHUNK_COLS, CHUNK_COLS)],
            buf.at[:, pl.ds(c * CHUNK_COLS, CHUNK_COLS)],
            insem.at[c],
        ).start()
    for c in range(NCHUNK):
        pltpu.make_async_copy(
            in_hbm.at[:, pl.ds(c * CHUNK_COLS, CHUNK_COLS)],
            buf.at[:, pl.ds(c * CHUNK_COLS, CHUNK_COLS)],
            insem.at[c],
        ).wait()
        pltpu.make_async_copy(
            buf.at[:, pl.ds(c * CHUNK_COLS, CHUNK_COLS)],
            out_hbm.at[:, pl.ds(c * CHUNK_COLS, CHUNK_COLS)],
            outsem.at[c],
        ).start()
    for c in range(NCHUNK):
        pltpu.make_async_copy(
            buf.at[:, pl.ds(c * CHUNK_COLS, CHUNK_COLS)],
            out_hbm.at[:, pl.ds(c * CHUNK_COLS, CHUNK_COLS)],
            outsem.at[c],
        ).wait()


def kernel(node_table):
    return pl.pallas_call(
        _dma_body,
        out_shape=jax.ShapeDtypeStruct((NODE_NUM, HIDDEN_SIZE), jnp.float32),
        in_specs=[pl.BlockSpec(memory_space=pl.ANY)],
        out_specs=pl.BlockSpec(memory_space=pl.ANY),
        scratch_shapes=[
            pltpu.VMEM((NODE_NUM, HIDDEN_SIZE), jnp.float32),
            pltpu.SemaphoreType.DMA((NCHUNK,)),
            pltpu.SemaphoreType.DMA((NCHUNK,)),
        ],
    )(node_table)
